# baseline (speedup 1.0000x reference)
import jax
import jax.numpy as jnp
from jax import lax
from jax.experimental import pallas as pl
from jax.experimental.pallas import tpu as pltpu
from jax.experimental.pallas import tpu_sc as plsc

VOCAB = 1000000
EMBED = 64
MAX_SEQ = 200
BATCH = 4096

NUM_CORES = 2
NUM_SUBCORES = 16
PLANES_PER_CORE = EMBED // NUM_CORES
BATCH_PER_TILE = BATCH // NUM_SUBCORES
HALF = BATCH_PER_TILE // 2
PLANE_SLICE = 62496
STAGE_TAIL = VOCAB - NUM_SUBCORES * PLANE_SLICE
LANES = 16
BLK = 40
BLOCKS_PER_HALF = MAX_SEQ // BLK
BLOCKS_PER_PLANE = 2 * BLOCKS_PER_HALF


def _body(xT_hbm, weT_hbm, pebc_hbm, outT_hbm,
          idx0, idx1, bufs, pe_v, plane, gsem, ssem, stgsem):
    c = lax.axis_index("c")
    t = lax.axis_index("s")
    bbase = t * BATCH_PER_TILE

    pltpu.sync_copy(xT_hbm.at[:, pl.ds(bbase, HALF)], idx0)
    pltpu.sync_copy(xT_hbm.at[:, pl.ds(bbase + HALF, HALF)], idx1)

    @pl.loop(0, PLANES_PER_CORE)
    def _(e_rel):
        e = c * PLANES_PER_CORE + e_rel

        pltpu.sync_copy(pebc_hbm.at[e], pe_v)

        pltpu.async_copy(
            weT_hbm.at[e, pl.ds(t * PLANE_SLICE, PLANE_SLICE)],
            plane.at[pl.ds(t * PLANE_SLICE, PLANE_SLICE)],
            stgsem,
        ).wait()

        @pl.when(t == 0)
        def _():
            pltpu.sync_copy(
                weT_hbm.at[e, pl.ds(VOCAB - STAGE_TAIL, STAGE_TAIL)],
                plane.at[pl.ds(VOCAB - STAGE_TAIL, STAGE_TAIL)],
            )

        plsc.subcore_barrier()

        for b in range(BLOCKS_PER_PLANE):
            h = b // BLOCKS_PER_HALF
            j = b % BLOCKS_PER_HALF
            r = b % 2
            idx = idx1 if h else idx0
            s0 = j * BLK
            col = bbase + h * HALF
            blkno = e_rel * BLOCKS_PER_PLANE + b

            @pl.when(blkno >= 2)
            def _():
                pltpu.make_async_copy(
                    bufs.at[r], outT_hbm.at[pl.ds(0, BLK), 0, pl.ds(col, HALF)],
                    ssem.at[r],
                ).wait()

            @pl.loop(s0, s0 + BLK)
            def _(s):
                pltpu.async_copy(
                    plane.at[idx.at[s]], bufs.at[r, s - s0], gsem.at[r]
                )

            pltpu.make_async_copy(
                outT_hbm.at[pl.ds(0, BLK), 0, pl.ds(col, HALF)], bufs.at[r],
                gsem.at[r],
            ).wait()

            @pl.loop(s0, s0 + BLK)
            def _(s):
                v = pe_v[pl.ds(s * LANES, LANES)]
                for c4 in range(HALF // LANES):
                    plsc.addupdate(bufs.at[r, s - s0, pl.ds(c4 * LANES, LANES)], v)

            pltpu.async_copy(
                bufs.at[r], outT_hbm.at[pl.ds(s0, BLK), e, pl.ds(col, HALF)],
                ssem.at[r],
            )

        plsc.subcore_barrier()

    for r in range(2):
        pltpu.make_async_copy(
            bufs.at[r], outT_hbm.at[pl.ds(0, BLK), 0, pl.ds(bbase, HALF)],
            ssem.at[r],
        ).wait()


@jax.jit
def _emb_kernel(xT, weT, pebc):
    mesh = plsc.VectorSubcoreMesh(
        core_axis_name="c", subcore_axis_name="s", num_cores=NUM_CORES,
        num_subcores=NUM_SUBCORES,
    )
    return pl.kernel(
        _body,
        out_type=jax.ShapeDtypeStruct((MAX_SEQ, EMBED, BATCH), jnp.float32),
        mesh=mesh,
        scratch_types=[
            pltpu.VMEM((MAX_SEQ, HALF), jnp.int32),
            pltpu.VMEM((MAX_SEQ, HALF), jnp.int32),
            pltpu.VMEM((2, BLK, HALF), jnp.float32),
            pltpu.VMEM((MAX_SEQ * LANES,), jnp.float32),
            pltpu.VMEM_SHARED((VOCAB,), jnp.float32),
            pltpu.SemaphoreType.DMA((2,)),
            pltpu.SemaphoreType.DMA((2,)),
            pltpu.SemaphoreType.DMA,
        ],
        compiler_params=pltpu.CompilerParams(use_tc_tiling_on_sc=False),
    )(xT, weT, pebc)


def kernel(x, we_table, pe_table):
    pebc = jnp.repeat(pe_table.T[:, :, None], LANES, axis=2).reshape(
        EMBED, MAX_SEQ * LANES)
    outT = _emb_kernel(x.T, we_table.T, pebc)
    return outT.transpose(2, 0, 1)


# speedup vs baseline: 4.3307x; 4.3307x over previous
import jax
import jax.numpy as jnp
from jax import lax
from jax.experimental import pallas as pl
from jax.experimental.pallas import tpu as pltpu
from jax.experimental.pallas import tpu_sc as plsc

VOCAB = 1000000
EMBED = 64
MAX_SEQ = 200
BATCH = 4096

NUM_CORES = 2
NUM_SUBCORES = 16
NUM_WORKERS = NUM_CORES * NUM_SUBCORES
ROWS_PER_WORKER = BATCH // NUM_WORKERS
PAIR = 2
PAIRS_PER_WORKER = ROWS_PER_WORKER // PAIR
IDX_SPLITS = ((0, 128), (128, 72))
BTILE = 512
STILE = 8


def _sc_body(x_hbm, we_hbm, out_hbm, idx_v, rows_v, gsem, ssem):
    wid = lax.axis_index("s") * NUM_CORES + lax.axis_index("c")
    row_base = wid * ROWS_PER_WORKER

    pltpu.sync_copy(x_hbm.at[pl.ds(row_base, ROWS_PER_WORKER)], idx_v)

    def fire_gathers(slot, p):
        for k in range(PAIR):
            for off, ln in IDX_SPLITS:
                pltpu.async_copy(
                    we_hbm.at[idx_v.at[PAIR * p + k, pl.ds(off, ln)]],
                    rows_v.at[slot, k, pl.ds(off, ln)],
                    gsem.at[slot],
                )

    def wait_gathers(slot):
        for k in range(PAIR):
            pltpu.make_async_copy(
                we_hbm.at[pl.ds(0, MAX_SEQ)], rows_v.at[slot, k], gsem.at[slot]
            ).wait()

    def fire_store(slot, p):
        pltpu.async_copy(
            rows_v.at[slot],
            out_hbm.at[pl.ds(row_base + PAIR * p, PAIR)],
            ssem.at[slot],
        )

    def wait_store(slot):
        pltpu.make_async_copy(
            rows_v.at[slot], out_hbm.at[pl.ds(0, PAIR)], ssem.at[slot]
        ).wait()

    fire_gathers(0, 0)

    @pl.loop(0, PAIRS_PER_WORKER, step=2)
    def _(p0):
        for q in range(2):
            p = p0 + q
            s = q
            o = 1 - q

            @pl.when(p >= 1)
            def _():
                wait_store(o)

            @pl.when(p + 1 < PAIRS_PER_WORKER)
            def _():
                fire_gathers(o, p + 1)

            wait_gathers(s)
            fire_store(s, p)

    wait_store(1)


def _sc_gather(x, we_table):
    mesh = plsc.VectorSubcoreMesh(
        core_axis_name="c", subcore_axis_name="s", num_cores=NUM_CORES,
        num_subcores=NUM_SUBCORES,
    )
    return pl.kernel(
        _sc_body,
        out_type=jax.ShapeDtypeStruct((BATCH, MAX_SEQ, EMBED), jnp.float32),
        mesh=mesh,
        scratch_types=[
            pltpu.VMEM((ROWS_PER_WORKER, MAX_SEQ), jnp.int32),
            pltpu.VMEM((2, PAIR, MAX_SEQ, EMBED), jnp.float32),
            pltpu.SemaphoreType.DMA((2,)),
            pltpu.SemaphoreType.DMA((2,)),
        ],
        compiler_params=pltpu.CompilerParams(use_tc_tiling_on_sc=False),
    )(x, we_table)


def _tc_body(rows_ref, pe_ref, out_ref):
    for si in range(STILE):
        pe_col = pe_ref[si, :][:, None]
        out_ref[si] = rows_ref[:, si, :].T + pe_col


def _tc_fixup(scratch, pe_table):
    grid = (MAX_SEQ // STILE, BATCH // BTILE)
    return pl.pallas_call(
        _tc_body,
        grid=grid,
        in_specs=[
            pl.BlockSpec((BTILE, STILE, EMBED), lambda s, b: (b, s, 0)),
            pl.BlockSpec((STILE, EMBED), lambda s, b: (s, 0)),
        ],
        out_specs=pl.BlockSpec((STILE, EMBED, BTILE), lambda s, b: (s, 0, b)),
        out_shape=jax.ShapeDtypeStruct((MAX_SEQ, EMBED, BATCH), jnp.float32),
    )(scratch, pe_table)


@jax.jit
def _emb_kernel(x, we_table, pe_table):
    scratch = _sc_gather(x, we_table)
    outT = _tc_fixup(scratch, pe_table)
    return outT.transpose(2, 0, 1)


def kernel(x, we_table, pe_table):
    return _emb_kernel(x, we_table, pe_table)


# speedup vs baseline: 4.4267x; 1.0222x over previous
import jax
import jax.numpy as jnp
from jax import lax
from jax.experimental import pallas as pl
from jax.experimental.pallas import tpu as pltpu
from jax.experimental.pallas import tpu_sc as plsc

VOCAB = 1000000
EMBED = 64
MAX_SEQ = 200
BATCH = 4096

NUM_CORES = 2
NUM_SUBCORES = 16
NUM_WORKERS = NUM_CORES * NUM_SUBCORES
ROWS_PER_WORKER = BATCH // NUM_WORKERS
PAIR = 2
PAIRS_PER_WORKER = ROWS_PER_WORKER // PAIR
IDX_SPLITS = ((0, 128), (128, 72))
BTILE = 512
STILE = 8


def _sc_body(x_hbm, we_hbm, out_hbm, idx_v, rows_v, gsem, ssem):
    wid = lax.axis_index("s") * NUM_CORES + lax.axis_index("c")
    row_base = wid * ROWS_PER_WORKER

    pltpu.sync_copy(x_hbm.at[pl.ds(row_base, ROWS_PER_WORKER)], idx_v)

    def fire_gathers(slot, p):
        for k in range(PAIR):
            for off, ln in IDX_SPLITS:
                pltpu.async_copy(
                    we_hbm.at[idx_v.at[PAIR * p + k, pl.ds(off, ln)]],
                    rows_v.at[slot, k, pl.ds(off, ln)],
                    gsem.at[slot],
                )

    def wait_gathers(slot):
        for k in range(PAIR):
            pltpu.make_async_copy(
                we_hbm.at[pl.ds(0, MAX_SEQ)], rows_v.at[slot, k], gsem.at[slot]
            ).wait()

    def fire_store(slot, p):
        pltpu.async_copy(
            rows_v.at[slot],
            out_hbm.at[pl.ds(row_base + PAIR * p, PAIR)],
            ssem.at[slot],
        )

    def wait_store(slot):
        pltpu.make_async_copy(
            rows_v.at[slot], out_hbm.at[pl.ds(0, PAIR)], ssem.at[slot]
        ).wait()

    fire_gathers(0, 0)

    @pl.loop(0, PAIRS_PER_WORKER, step=2)
    def _(p0):
        for q in range(2):
            p = p0 + q
            s = q
            o = 1 - q

            @pl.when(p >= 1)
            def _():
                wait_store(o)

            @pl.when(p + 1 < PAIRS_PER_WORKER)
            def _():
                fire_gathers(o, p + 1)

            wait_gathers(s)
            fire_store(s, p)

    wait_store(1)


def _sc_gather(x, we_table):
    mesh = plsc.VectorSubcoreMesh(
        core_axis_name="c", subcore_axis_name="s", num_cores=NUM_CORES,
        num_subcores=NUM_SUBCORES,
    )
    return pl.kernel(
        _sc_body,
        out_type=jax.ShapeDtypeStruct((BATCH, MAX_SEQ, EMBED), jnp.float32),
        mesh=mesh,
        scratch_types=[
            pltpu.VMEM((ROWS_PER_WORKER, MAX_SEQ), jnp.int32),
            pltpu.VMEM((2, PAIR, MAX_SEQ, EMBED), jnp.float32),
            pltpu.SemaphoreType.DMA((2,)),
            pltpu.SemaphoreType.DMA((2,)),
        ],
        compiler_params=pltpu.CompilerParams(use_tc_tiling_on_sc=False),
    )(x, we_table)


def _tc_body(rows_ref, pe_ref, out_ref):
    rows = rows_ref[...].reshape(BTILE, STILE * EMBED)
    out = rows.T.reshape(STILE, EMBED, BTILE)
    out_ref[...] = out + pe_ref[...][:, :, None]


def _tc_fixup(scratch, pe_table):
    grid = (MAX_SEQ // STILE, BATCH // BTILE)
    return pl.pallas_call(
        _tc_body,
        grid=grid,
        in_specs=[
            pl.BlockSpec((BTILE, STILE, EMBED), lambda s, b: (b, s, 0)),
            pl.BlockSpec((STILE, EMBED), lambda s, b: (s, 0)),
        ],
        out_specs=pl.BlockSpec((STILE, EMBED, BTILE), lambda s, b: (s, 0, b)),
        out_shape=jax.ShapeDtypeStruct((MAX_SEQ, EMBED, BATCH), jnp.float32),
    )(scratch, pe_table)


@jax.jit
def _emb_kernel(x, we_table, pe_table):
    scratch = _sc_gather(x, we_table)
    outT = _tc_fixup(scratch, pe_table)
    return outT.transpose(2, 0, 1)


def kernel(x, we_table, pe_table):
    return _emb_kernel(x, we_table, pe_table)


# speedup vs baseline: 4.8485x; 1.0953x over previous
import jax
import jax.numpy as jnp
from jax import lax
from jax.experimental import pallas as pl
from jax.experimental.pallas import tpu as pltpu
from jax.experimental.pallas import tpu_sc as plsc

VOCAB = 1000000
EMBED = 64
MAX_SEQ = 200
BATCH = 4096

NUM_CORES = 2
NUM_SUBCORES = 16
NUM_WORKERS = NUM_CORES * NUM_SUBCORES
ROWS_PER_WORKER = BATCH // NUM_WORKERS
PAIR = 2
PAIRS_PER_WORKER = ROWS_PER_WORKER // PAIR
IDX_SPLITS = ((0, 128), (128, 72))
LANES = 16


def _body(x_hbm, we_hbm, pe_hbm, out_hbm, idx_v, rows_v, pe_v, gsem, ssem):
    wid = lax.axis_index("s") * NUM_CORES + lax.axis_index("c")
    row_base = wid * ROWS_PER_WORKER

    pltpu.sync_copy(pe_hbm, pe_v)
    pltpu.sync_copy(x_hbm.at[pl.ds(row_base, ROWS_PER_WORKER)], idx_v)

    def fire_gathers(slot, p):
        for k in range(PAIR):
            for off, ln in IDX_SPLITS:
                pltpu.async_copy(
                    we_hbm.at[idx_v.at[PAIR * p + k, pl.ds(off, ln)]],
                    rows_v.at[slot, k, pl.ds(off, ln)],
                    gsem.at[slot],
                )

    def wait_gathers(slot):
        for k in range(PAIR):
            pltpu.make_async_copy(
                we_hbm.at[pl.ds(0, MAX_SEQ)], rows_v.at[slot, k], gsem.at[slot]
            ).wait()

    def fire_store(slot, p):
        pltpu.async_copy(
            rows_v.at[slot],
            out_hbm.at[pl.ds(row_base + PAIR * p, PAIR)],
            ssem.at[slot],
        )

    def wait_store(slot):
        pltpu.make_async_copy(
            rows_v.at[slot], out_hbm.at[pl.ds(0, PAIR)], ssem.at[slot]
        ).wait()

    def add_pe(slot):
        @pl.loop(0, MAX_SEQ)
        def _(r):
            for c in range(EMBED // LANES):
                sl = pl.ds(c * LANES, LANES)
                v = pe_v[r, sl]
                plsc.addupdate(rows_v.at[slot, 0, r, sl], v)
                plsc.addupdate(rows_v.at[slot, 1, r, sl], v)

    fire_gathers(0, 0)

    @pl.loop(0, PAIRS_PER_WORKER, step=2)
    def _(p0):
        for q in range(2):
            p = p0 + q
            s = q
            o = 1 - q

            @pl.when(p >= 1)
            def _():
                wait_store(o)

            @pl.when(p + 1 < PAIRS_PER_WORKER)
            def _():
                fire_gathers(o, p + 1)

            wait_gathers(s)
            add_pe(s)
            fire_store(s, p)

    wait_store(1)


@jax.jit
def _emb_kernel(x, we_table, pe_table):
    mesh = plsc.VectorSubcoreMesh(
        core_axis_name="c", subcore_axis_name="s", num_cores=NUM_CORES,
        num_subcores=NUM_SUBCORES,
    )
    return pl.kernel(
        _body,
        out_type=jax.ShapeDtypeStruct((BATCH, MAX_SEQ, EMBED), jnp.float32),
        mesh=mesh,
        scratch_types=[
            pltpu.VMEM((ROWS_PER_WORKER, MAX_SEQ), jnp.int32),
            pltpu.VMEM((2, PAIR, MAX_SEQ, EMBED), jnp.float32),
            pltpu.VMEM((MAX_SEQ, EMBED), jnp.float32),
            pltpu.SemaphoreType.DMA((2,)),
            pltpu.SemaphoreType.DMA((2,)),
        ],
        compiler_params=pltpu.CompilerParams(use_tc_tiling_on_sc=False),
    )(x, we_table, pe_table)


def kernel(x, we_table, pe_table):
    return _emb_kernel(x, we_table, pe_table)
